# baseline (speedup 1.0000x reference)
import functools

import jax
import jax.numpy as jnp
from jax import lax
from jax.experimental import pallas as pl
from jax.experimental.pallas import tpu as pltpu
from jax.experimental.pallas import tpu_sc as plsc

DIM = 32
NUM_HEADS = 8
ROW = NUM_HEADS * DIM * DIM
CH = 4


def _sc_gather(table2d, idx2d):
    n_rows_out = idx2d.shape[0] * idx2d.shape[1]
    info = plsc.get_sparse_core_info()
    nc, ns = info.num_cores, info.num_subcores
    nw = nc * ns
    b_per_w = n_rows_out // nw
    n_chunks = b_per_w // CH

    mesh = plsc.VectorSubcoreMesh(core_axis_name="c", subcore_axis_name="s")

    @functools.partial(
        pl.kernel,
        mesh=mesh,
        out_type=jax.ShapeDtypeStruct((n_rows_out, ROW), jnp.float32),
        scratch_types=[
            pltpu.VMEM((n_chunks, CH), jnp.int32),
            pltpu.VMEM((CH, ROW), jnp.float32),
            pltpu.VMEM((CH, ROW), jnp.float32),
            pltpu.VMEM((CH, ROW), jnp.float32),
            pltpu.SemaphoreType.DMA,
            pltpu.SemaphoreType.DMA,
            pltpu.SemaphoreType.DMA,
            pltpu.SemaphoreType.DMA,
            pltpu.SemaphoreType.DMA,
            pltpu.SemaphoreType.DMA,
        ],
    )
    def k(table_hbm, idx_hbm, out_hbm, idx_v,
          buf0, buf1, buf2, g0, g1, g2, s0, s1, s2):
        wid = lax.axis_index("s") * nc + lax.axis_index("c")
        base = wid * b_per_w
        pltpu.sync_copy(idx_hbm.at[pl.ds(wid * n_chunks, n_chunks)], idx_v)

        bufs = (buf0, buf1, buf2)
        gsems = (g0, g1, g2)
        ssems = (s0, s1, s2)

        def gather_start(i, b):
            pltpu.make_async_copy(
                table_hbm.at[idx_v.at[i]], bufs[b], gsems[b]).start()

        def gather_wait(b):
            pltpu.make_async_copy(
                table_hbm.at[idx_v.at[0]], bufs[b], gsems[b]).wait()

        def scatter_start(i, b):
            pltpu.make_async_copy(
                bufs[b], out_hbm.at[pl.ds(base + i * CH, CH)], ssems[b]).start()

        def scatter_wait(b):
            pltpu.make_async_copy(
                bufs[b], out_hbm.at[pl.ds(base, CH)], ssems[b]).wait()

        n = n_chunks
        gather_start(0, 0)
        gather_start(1, 1)
        gather_start(2, 2)
        gather_wait(0)
        scatter_start(0, 0)
        scatter_wait(0)
        gather_start(3, 0)
        gather_wait(1)
        scatter_start(1, 1)

        def loop_body(t, carry):
            for u in range(3):
                i = 3 * t + 2 + u
                bw = (1 + u) % 3
                bc = (2 + u) % 3
                scatter_wait(bw)
                gather_start(i + 2, bw)
                gather_wait(bc)
                scatter_start(i, bc)
            return carry

        lax.fori_loop(0, (n - 4) // 3, loop_body, 0)

        bw = (n - 2) % 3
        scatter_wait((n - 3) % 3)
        gather_wait(bw)
        scatter_start(n - 2, bw)
        scatter_wait((n - 2) % 3)
        gather_wait((n - 1) % 3)
        scatter_start(n - 1, (n - 1) % 3)
        scatter_wait((n - 1) % 3)

    return k


def kernel(mapping, maps):
    idx2d = jnp.ravel(mapping).astype(jnp.int32).reshape(-1, CH)
    table2d = maps.reshape(maps.shape[0], -1)
    out = _sc_gather(table2d, idx2d)(table2d, idx2d)
    return out.reshape(tuple(mapping.shape) + (NUM_HEADS, DIM, DIM))


# speedup vs baseline: 1.8214x; 1.8214x over previous
import functools

import jax
import jax.numpy as jnp
from jax import lax
from jax.experimental import pallas as pl
from jax.experimental.pallas import tpu as pltpu
from jax.experimental.pallas import tpu_sc as plsc

DIM = 32
NUM_HEADS = 8
NFEAT = NUM_HEADS * DIM * DIM
NPOS = 4096
RSLAB = 4


def _sc_gather(tableT, mapping):
    nb, seq = mapping.shape
    info = plsc.get_sparse_core_info()
    nc, ns = info.num_cores, info.num_subcores
    nw = nc * ns
    f_per_w = NFEAT // nw
    n_groups = f_per_w // RSLAB
    n_vec = seq // 16

    mesh = plsc.VectorSubcoreMesh(core_axis_name="c", subcore_axis_name="s")

    @functools.partial(
        pl.kernel,
        mesh=mesh,
        compiler_params=pltpu.CompilerParams(needs_layout_passes=False),
        out_type=jax.ShapeDtypeStruct((nb, NFEAT, seq), jnp.float32),
        scratch_types=[
            pltpu.VMEM((nb, seq), jnp.int32),
            pltpu.VMEM((RSLAB, NPOS), jnp.float32),
            pltpu.VMEM((RSLAB, NPOS), jnp.float32),
            pltpu.VMEM((nb, RSLAB, seq), jnp.float32),
            pltpu.VMEM((nb, RSLAB, seq), jnp.float32),
            pltpu.SemaphoreType.DMA,
            pltpu.SemaphoreType.DMA,
            pltpu.SemaphoreType.DMA,
            pltpu.SemaphoreType.DMA,
        ],
    )
    def k(table_hbm, idx_hbm, out_hbm, idx_v,
          slab0, slab1, ob0, ob1, i0, i1, o0, o1):
        wid = lax.axis_index("s") * nc + lax.axis_index("c")
        base_f = wid * f_per_w
        pltpu.sync_copy(idx_hbm, idx_v)

        slabs = (slab0, slab1)
        obufs = (ob0, ob1)
        isems = (i0, i1)
        osems = (o0, o1)

        def in_start(g, p):
            pltpu.make_async_copy(
                table_hbm.at[pl.ds(base_f + g * RSLAB, RSLAB)],
                slabs[p], isems[p]).start()

        def in_wait(p):
            pltpu.make_async_copy(
                table_hbm.at[pl.ds(base_f, RSLAB)], slabs[p], isems[p]).wait()

        def out_start(g, p):
            for b in range(nb):
                pltpu.make_async_copy(
                    obufs[p].at[pl.ds(b, 1)],
                    out_hbm.at[pl.ds(b, 1), pl.ds(base_f + g * RSLAB, RSLAB)],
                    osems[p]).start()

        def out_drain(p):
            for b in range(nb):
                pltpu.make_async_copy(
                    obufs[p].at[pl.ds(b, 1)],
                    out_hbm.at[pl.ds(0, 1), pl.ds(base_f, RSLAB)],
                    osems[p]).wait()

        row_ids = tuple(
            jnp.full((16,), r, dtype=jnp.int32) for r in range(RSLAB))

        def compute(p):
            slab = slabs[p]
            obuf = obufs[p]

            def body(v, carry):
                off = v * 16
                for b in range(nb):
                    idxv = idx_v[b, pl.ds(off, 16)]
                    for r in range(RSLAB):
                        obuf[b, r, pl.ds(off, 16)] = plsc.load_gather(
                            slab, [row_ids[r], idxv])
                return carry

            lax.fori_loop(0, n_vec, body, 0)

        in_start(0, 0)
        in_start(1, 1)
        in_wait(0)
        compute(0)
        out_start(0, 0)
        in_start(2, 0)
        in_wait(1)
        compute(1)
        out_start(1, 1)
        in_start(3, 1)

        def loop_body(t, carry):
            for u in range(2):
                g = 2 * t + 2 + u
                in_wait(u)
                out_drain(u)
                compute(u)
                out_start(g, u)
                in_start(g + 2, u)
            return carry

        lax.fori_loop(0, (n_groups - 4) // 2, loop_body, 0)

        for u in range(2):
            in_wait(u)
            out_drain(u)
            compute(u)
            out_start(n_groups - 2 + u, u)
        out_drain(0)
        out_drain(1)

    return k


def kernel(mapping, maps):
    tableT = maps.transpose(1, 2, 3, 0).reshape(NFEAT, NPOS)
    idx = mapping.astype(jnp.int32)
    out = _sc_gather(tableT, idx)(tableT, idx)
    nb, seq = mapping.shape
    return out.reshape(nb, NUM_HEADS, DIM, DIM, seq).transpose(0, 4, 1, 2, 3)
